# baseline (speedup 1.0000x reference)
import jax
import jax.numpy as jnp
from jax import lax
from jax.experimental import pallas as pl
from jax.experimental.pallas import tpu as pltpu
from jax.experimental.pallas import tpu_sc as plsc

_ALT_VALUES = (150, 200, 250, 300)
_NUM_PARTS = 16
_NUM_ROWS = 5
_BATCH = 16384
_NC, _NS = 2, 16
_NW = _NC * _NS
_BPW = _BATCH // _NW
_GROUPS = _BPW // 16
_FROWS = _BATCH // 8


def _tc_prefill_kernel(att_ref, temp_ref, table_ref, fill_ref):
    t = jnp.maximum(jnp.abs(temp_ref[0, 0]), jnp.float32(0.1))
    w = att_ref[...] / t
    m = jnp.max(w, axis=-1, keepdims=True)
    e = jnp.exp(w - m)
    sm = e / jnp.sum(e, axis=-1, keepdims=True)
    table_ref[...] = sm
    wide = jnp.concatenate([sm[_NUM_ROWS - 1:_NUM_ROWS, :]] * 8, axis=1)
    fill_ref[...] = jnp.broadcast_to(wide, fill_ref.shape)


def _tc_prefill(attention, temp):
    return pl.pallas_call(
        _tc_prefill_kernel,
        out_shape=(
            jax.ShapeDtypeStruct((_NUM_ROWS, _NUM_PARTS), jnp.float32),
            jax.ShapeDtypeStruct((_FROWS, 128), jnp.float32),
        ),
        in_specs=[
            pl.BlockSpec(memory_space=pltpu.VMEM),
            pl.BlockSpec(memory_space=pltpu.SMEM),
        ],
    )(attention, temp.reshape(1, 1))


def _sc_fixup_kernel(table_hbm, alt_hbm, out_hbm, table_v, alt_v, sem):
    wid = lax.axis_index("s") * _NC + lax.axis_index("c")
    base = wid * _BPW
    pltpu.sync_copy(table_hbm, table_v)
    pltpu.sync_copy(alt_hbm.at[pl.ds(base, _BPW)], alt_v)

    def group_body(gg, cnt):
        a = alt_v[pl.ds(gg * 16, 16)]
        idx = jnp.full((16,), _NUM_ROWS - 1, dtype=jnp.int32)
        for i, v in enumerate(_ALT_VALUES):
            idx = jnp.where(a == jnp.int32(v), jnp.int32(i), idx)
        nhit = jnp.sum(jnp.where(idx != _NUM_ROWS - 1, 1, 0).astype(jnp.int32))

        @pl.when(nhit > 0)
        def _fixup():
            for k in range(16):
                ik = idx[k]

                @pl.when(ik != _NUM_ROWS - 1)
                def _one(ik=ik, k=k):
                    r = base + gg * 16 + k
                    pltpu.async_copy(
                        table_v.at[ik],
                        out_hbm.at[r >> 3, pl.ds((r & 7) * _NUM_PARTS,
                                                 _NUM_PARTS)],
                        sem,
                    )
        return cnt + nhit

    total = lax.fori_loop(0, _GROUPS, group_body, jnp.int32(0))

    def drain_body(i, carry):
        pltpu.make_async_copy(
            table_v.at[0],
            out_hbm.at[0, pl.ds(0, _NUM_PARTS)],
            sem).wait()
        return carry

    lax.fori_loop(0, total, drain_body, 0)


def kernel(altitudes, attention, temp):
    table, filled = _tc_prefill(attention, temp)
    out_ref = jax.new_ref(filled)
    mesh = plsc.VectorSubcoreMesh(core_axis_name="c", subcore_axis_name="s")
    run = pl.kernel(
        _sc_fixup_kernel,
        out_type=(),
        mesh=mesh,
        compiler_params=pltpu.CompilerParams(
            use_tc_tiling_on_sc=True, needs_layout_passes=False),
        scratch_types=[
            pltpu.VMEM((_NUM_ROWS, _NUM_PARTS), jnp.float32),
            pltpu.VMEM((_BPW,), jnp.int32),
            pltpu.SemaphoreType.DMA,
        ],
    )
    run(table, altitudes, out_ref)
    return jax.freeze(out_ref).reshape(_BATCH, _NUM_PARTS)


# speedup vs baseline: 1.1283x; 1.1283x over previous
import jax
import jax.numpy as jnp
from jax import lax
from jax.experimental import pallas as pl
from jax.experimental.pallas import tpu as pltpu
from jax.experimental.pallas import tpu_sc as plsc

_ALT_VALUES = (150, 200, 250, 300)
_NUM_PARTS = 16
_NUM_ROWS = 5
_BATCH = 16384
_NC, _NS = 2, 16
_NW = _NC * _NS
_BPW = _BATCH // _NW
_GROUPS = _BPW // 16


_NCHUNK = 8
_ROWS_PER_CHUNK = _BPW // _NCHUNK
_GROUPS_PER_CHUNK = _ROWS_PER_CHUNK // 16


def _sc_kernel(att_hbm, alt_hbm, temp_hbm, out_hbm, att_v, temp_v, alt_v,
               table_v, rows_v, sems):
    wid = lax.axis_index("s") * _NC + lax.axis_index("c")
    base = wid * _BPW
    pltpu.sync_copy(att_hbm, att_v)
    pltpu.sync_copy(temp_hbm, temp_v)
    pltpu.sync_copy(alt_hbm.at[pl.ds(base, _BPW)], alt_v)

    recip = 1.0 / jnp.maximum(jnp.abs(temp_v[...]), jnp.float32(0.1))
    for i in range(_NUM_ROWS):
        w = att_v[i] * recip
        e = jnp.exp(w - jnp.max(w))
        table_v[pl.ds(i * _NUM_PARTS, _NUM_PARTS)] = e / jnp.sum(e)

    default_row = table_v[pl.ds(4 * _NUM_PARTS, _NUM_PARTS)]

    def chunk_body(c, carry):
        def fill_body(e, carry2):
            rows_v[c * _ROWS_PER_CHUNK + e, :] = default_row
            return carry2

        lax.fori_loop(0, _ROWS_PER_CHUNK, fill_body, 0, unroll=4)

        def group_body(gg, carry2):
            g = c * _GROUPS_PER_CHUNK + gg
            a = alt_v[pl.ds(g * 16, 16)]
            hits = (a == jnp.int32(_ALT_VALUES[0]))
            for v in _ALT_VALUES[1:]:
                hits = hits | (a == jnp.int32(v))

            @pl.when(jnp.any(hits))
            def _fixup():
                idx = jnp.full((16,), 4, dtype=jnp.int32)
                for i, v in enumerate(_ALT_VALUES):
                    idx = jnp.where(a == jnp.int32(v), jnp.int32(i), idx)
                offs = idx * _NUM_PARTS
                for k in range(16):
                    rows_v[g * 16 + k, :] = table_v[pl.ds(offs[k],
                                                          _NUM_PARTS)]
            return carry2

        lax.fori_loop(0, _GROUPS_PER_CHUNK, group_body, 0)
        pltpu.async_copy(
            rows_v.at[pl.ds(c * _ROWS_PER_CHUNK, _ROWS_PER_CHUNK)],
            out_hbm.at[pl.ds(base + c * _ROWS_PER_CHUNK, _ROWS_PER_CHUNK)],
            sems,
        )
        return carry

    lax.fori_loop(0, _NCHUNK, chunk_body, 0)
    for _ in range(_NCHUNK):
        pltpu.make_async_copy(
            rows_v.at[pl.ds(0, _ROWS_PER_CHUNK)],
            out_hbm.at[pl.ds(base, _ROWS_PER_CHUNK)],
            sems,
        ).wait()


def kernel(altitudes, attention, temp):
    mesh = plsc.VectorSubcoreMesh(core_axis_name="c", subcore_axis_name="s")
    run = pl.kernel(
        _sc_kernel,
        out_type=jax.ShapeDtypeStruct((_BATCH, _NUM_PARTS), jnp.float32),
        mesh=mesh,
        compiler_params=pltpu.CompilerParams(
            use_tc_tiling_on_sc=True, needs_layout_passes=False),
        scratch_types=[
            pltpu.VMEM((_NUM_ROWS, _NUM_PARTS), jnp.float32),
            pltpu.VMEM((16,), jnp.float32),
            pltpu.VMEM((_BPW,), jnp.int32),
            pltpu.VMEM((_NUM_ROWS * _NUM_PARTS,), jnp.float32),
            pltpu.VMEM((_BPW, _NUM_PARTS), jnp.float32),
            pltpu.SemaphoreType.DMA,
        ],
    )
    temp16 = jnp.broadcast_to(jnp.asarray(temp, jnp.float32).reshape(1), (16,))
    return run(attention, altitudes, temp16)
